# Initial kernel scaffold; baseline (speedup 1.0000x reference)
import jax
import jax.numpy as jnp
from jax.experimental import pallas as pl


def kernel(mem, label_mem, idx, val, label_val):
    raise NotImplementedError("write your pallas kernel here")


# speedup vs baseline: 1.1782x; 1.1782x over previous
import functools

import jax
import jax.numpy as jnp
from jax import lax
from jax.experimental import pallas as pl
from jax.experimental.pallas import tpu as pltpu
from jax.experimental.pallas import tpu_sc as plsc

MEM = 16384
D = 3 * 32 * 32
BATCH = 1024
ROWS_PER_BLK = 512
NBLK = MEM // ROWS_PER_BLK
NW = 32
B_PER = BATCH // NW
LANES = 16


def _tc_prep_body(mem_in, lab_in, idxa, idxb, lvb, mem_out, lab_out, zeros_out,
                  win_out, labscat_out):
    mem_out[...] = mem_in[...]
    lab_out[...] = lab_in[...]
    zeros_out[...] = jnp.zeros_like(zeros_out)

    @pl.when(pl.program_id(0) == 0)
    def _():
        a = idxa[...]
        b = idxb[...]
        lv = lvb[...]
        kk = lax.broadcasted_iota(jnp.int32, (BATCH, BATCH), 1)
        code = jnp.where(a == b, kk * 256 + lv, -1)
        best = jnp.max(code, axis=1, keepdims=True)
        win_out[...] = best >> 8
        labscat_out[...] = best & 255


_tc_prep = pl.pallas_call(
    _tc_prep_body,
    grid=(NBLK,),
    in_specs=[
        pl.BlockSpec((ROWS_PER_BLK, D), lambda i: (i, 0)),
        pl.BlockSpec((1, 1, ROWS_PER_BLK), lambda i: (i, 0, 0)),
        pl.BlockSpec((BATCH, 1), lambda i: (0, 0)),
        pl.BlockSpec((1, BATCH), lambda i: (0, 0)),
        pl.BlockSpec((1, BATCH), lambda i: (0, 0)),
    ],
    out_specs=[
        pl.BlockSpec((ROWS_PER_BLK, D), lambda i: (i, 0)),
        pl.BlockSpec((1, 1, ROWS_PER_BLK), lambda i: (i, 0, 0)),
        pl.BlockSpec((1, 1, ROWS_PER_BLK), lambda i: (i, 0, 0)),
        pl.BlockSpec((BATCH, 1), lambda i: (0, 0)),
        pl.BlockSpec((BATCH, 1), lambda i: (0, 0)),
    ],
    out_shape=[
        jax.ShapeDtypeStruct((MEM, D), jnp.float32),
        jax.ShapeDtypeStruct((NBLK, 1, ROWS_PER_BLK), jnp.int32),
        jax.ShapeDtypeStruct((NBLK, 1, ROWS_PER_BLK), jnp.int32),
        jax.ShapeDtypeStruct((BATCH, 1), jnp.int32),
        jax.ShapeDtypeStruct((BATCH, 1), jnp.int32),
    ],
)

_sc_mesh = plsc.VectorSubcoreMesh(core_axis_name="c", subcore_axis_name="s")


@functools.partial(
    pl.kernel,
    mesh=_sc_mesh,
    out_type=(),
    scratch_types=[
        pltpu.VMEM((B_PER,), jnp.int32),
        pltpu.VMEM((B_PER,), jnp.int32),
        pltpu.VMEM((B_PER,), jnp.int32),
        pltpu.VMEM((B_PER, D), jnp.float32),
        pltpu.SemaphoreType.DMA,
    ],
)
def _sc_scatter(mem_ref, lab_ref, idx_hbm, win_hbm, labscat_hbm, val_hbm,
                idx_v, win_v, labs_v, rows_v, sem):
    wid = lax.axis_index("s") * 2 + lax.axis_index("c")
    base = wid * B_PER
    pltpu.sync_copy(idx_hbm.at[pl.ds(base, B_PER)], idx_v)
    pltpu.sync_copy(win_hbm.at[pl.ds(base, B_PER)], win_v)
    pltpu.sync_copy(labscat_hbm.at[pl.ds(base, B_PER)], labs_v)
    pltpu.async_copy(val_hbm.at[win_v], rows_v, sem).wait()
    pltpu.async_copy(rows_v, mem_ref.at[idx_v], sem).wait()
    pltpu.async_copy(labs_v, lab_ref.at[idx_v], sem).wait()


def kernel(mem, label_mem, idx, val, label_val):
    mem2 = mem.reshape(MEM, D)
    val2 = val.reshape(BATCH, D)
    idx32 = idx.astype(jnp.int32)
    lv32 = label_val.astype(jnp.int32)

    new_mem0, new_lab0, zeros3, win, labscat = _tc_prep(
        mem2,
        label_mem.astype(jnp.int32).reshape(NBLK, 1, ROWS_PER_BLK),
        idx32.reshape(BATCH, 1),
        idx32.reshape(1, BATCH),
        lv32.reshape(1, BATCH),
    )

    mem_ref = jax.new_ref(new_mem0)
    lab_ref = jax.new_ref(new_lab0.reshape(MEM))
    _sc_scatter(
        mem_ref,
        lab_ref,
        idx32,
        win.reshape(BATCH),
        labscat.reshape(BATCH),
        val2,
    )
    new_mem = jax.freeze(mem_ref).reshape(MEM, 3, 32, 32)
    new_label = jax.freeze(lab_ref)
    return new_mem, new_label, zeros3.reshape(MEM)
